# baseline (speedup 1.0000x reference)
import functools

import jax
import jax.numpy as jnp
from jax import lax
from jax.experimental import pallas as pl
from jax.experimental.pallas import tpu as pltpu
from jax.experimental.pallas import tpu_sc as plsc

HIDDEN = 768
NUM_EXPERTS = 64
TOP_K = 2
BT = 4096

T = 32768
NC, NS, L = 2, 16, 16
NW = NC * NS
TPW = T // NW
GROUPS = TPW // L


def _logits_block(x_ref, wt_ref, b_ref, logits_ref):
    logits_ref[...] = jax.lax.dot_general(
        x_ref[...], wt_ref[...], (((1,), (0,)), ((), ())),
        preferred_element_type=jnp.float32,
    ) + b_ref[...][None, :]


def _route_body(logits_hbm, w_hbm, e_hbm, chunk_v, w_v, e_v):
    cid = lax.axis_index("c")
    sid = lax.axis_index("s")
    wid = sid * NC + cid
    base = wid * TPW
    pltpu.sync_copy(
        logits_hbm.at[pl.ds(base * NUM_EXPERTS, TPW * NUM_EXPERTS)], chunk_v)

    lane = lax.iota(jnp.int32, L)

    def group(g, _):
        row = g * L + lane
        flat = row * NUM_EXPERTS
        neg = jnp.full((L,), -jnp.inf, jnp.float32)
        zero = jnp.zeros((L,), jnp.int32)

        def estep(e, carry):
            m1, m2, a1, a2 = carry
            ev = jnp.full((L,), e, jnp.int32)
            v = plsc.load_gather(chunk_v, [flat + ev])
            gt1 = v > m1
            gt2 = v > m2
            a2 = jnp.where(gt1, a1, jnp.where(gt2, ev, a2))
            m2 = jnp.where(gt1, m1, jnp.where(gt2, v, m2))
            a1 = jnp.where(gt1, ev, a1)
            m1 = jnp.where(gt1, v, m1)
            return (m1, m2, a1, a2)

        m1, m2, a1, a2 = lax.fori_loop(
            0, NUM_EXPERTS, estep, (neg, neg, zero, zero))
        w1 = 1.0 / (1.0 + jnp.exp(m2 - m1))
        w2 = 1.0 - w1
        pos = row * 2
        plsc.store_scatter(w_v, [pos], w1)
        plsc.store_scatter(w_v, [pos + 1], w2)
        plsc.store_scatter(e_v, [pos], a1)
        plsc.store_scatter(e_v, [pos + 1], a2)
        return 0

    lax.fori_loop(0, GROUPS, group, 0)
    pltpu.sync_copy(w_v, w_hbm.at[pl.ds(base * 2, 2 * TPW)])
    pltpu.sync_copy(e_v, e_hbm.at[pl.ds(base * 2, 2 * TPW)])


_route = pl.kernel(
    _route_body,
    out_type=[
        jax.ShapeDtypeStruct((2 * T,), jnp.float32),
        jax.ShapeDtypeStruct((2 * T,), jnp.int32),
    ],
    mesh=plsc.VectorSubcoreMesh(
        core_axis_name="c", subcore_axis_name="s",
        num_cores=NC, num_subcores=NS),
    scratch_types=[
        pltpu.VMEM((TPW * NUM_EXPERTS,), jnp.float32),
        pltpu.VMEM((2 * TPW,), jnp.float32),
        pltpu.VMEM((2 * TPW,), jnp.int32),
    ],
    compiler_params=pltpu.CompilerParams(needs_layout_passes=False),
)


@jax.jit
def kernel(hidden_states, gate_w, gate_b):
    b, s, h = hidden_states.shape
    t = b * s
    x = hidden_states.reshape(t, h)
    wt = gate_w.T

    logits = pl.pallas_call(
        _logits_block,
        grid=(t // BT,),
        in_specs=[
            pl.BlockSpec((BT, h), lambda i: (i, 0)),
            pl.BlockSpec((h, NUM_EXPERTS), lambda i: (0, 0)),
            pl.BlockSpec((NUM_EXPERTS,), lambda i: (0,)),
        ],
        out_specs=pl.BlockSpec((BT, NUM_EXPERTS), lambda i: (i, 0)),
        out_shape=jax.ShapeDtypeStruct((t, NUM_EXPERTS), jnp.float32),
    )(x, wt, gate_b)

    wf, ef = _route(logits.reshape(t * NUM_EXPERTS))
    weights = wf.reshape(t, TOP_K)
    experts = ef.reshape(t, TOP_K)
    aux_loss = jnp.array(0.0, dtype=jnp.float32)
    return (weights, experts, logits, aux_loss)


# speedup vs baseline: 1.1449x; 1.1449x over previous
import functools

import jax
import jax.numpy as jnp
from jax import lax
from jax.experimental import pallas as pl
from jax.experimental.pallas import tpu as pltpu
from jax.experimental.pallas import tpu_sc as plsc

HIDDEN = 768
NUM_EXPERTS = 64
TOP_K = 2
BT = 4096

T = 32768
NC, NS, L = 2, 16, 16
NW = NC * NS
TPW = T // NW
GROUPS = TPW // L


def _logits_block(x_ref, wt_ref, b_ref, logits_ref):
    logits_ref[...] = jax.lax.dot_general(
        x_ref[...], wt_ref[...], (((1,), (0,)), ((), ())),
        preferred_element_type=jnp.float32,
    ) + b_ref[...][None, :]


def _route_body(logits_hbm, w_hbm, e_hbm, chunk_v, w_v, e_v):
    cid = lax.axis_index("c")
    sid = lax.axis_index("s")
    wid = sid * NC + cid
    base = wid * TPW
    pltpu.sync_copy(
        logits_hbm.at[pl.ds(base * NUM_EXPERTS, TPW * NUM_EXPERTS)], chunk_v)

    lane = lax.iota(jnp.int32, L)

    def group(g, _):
        row = g * L + lane
        flat = row * NUM_EXPERTS
        neg = jnp.full((L,), -jnp.inf, jnp.float32)
        zero = jnp.zeros((L,), jnp.int32)

        m1, m2, a1, a2 = neg, neg, zero, zero
        for e in range(NUM_EXPERTS):
            ev = jnp.full((L,), e, jnp.int32)
            v = plsc.load_gather(chunk_v, [flat + e])
            gt1 = v > m1
            gt2 = v > m2
            a2 = jnp.where(gt1, a1, jnp.where(gt2, ev, a2))
            m2 = jnp.where(gt1, m1, jnp.where(gt2, v, m2))
            a1 = jnp.where(gt1, ev, a1)
            m1 = jnp.where(gt1, v, m1)
        w1 = 1.0 / (1.0 + jnp.exp(m2 - m1))
        w2 = 1.0 - w1
        pos = row * 2
        plsc.store_scatter(w_v, [pos], w1)
        plsc.store_scatter(w_v, [pos + 1], w2)
        plsc.store_scatter(e_v, [pos], a1)
        plsc.store_scatter(e_v, [pos + 1], a2)
        return 0

    lax.fori_loop(0, GROUPS, group, 0)
    pltpu.sync_copy(w_v, w_hbm.at[pl.ds(base * 2, 2 * TPW)])
    pltpu.sync_copy(e_v, e_hbm.at[pl.ds(base * 2, 2 * TPW)])


_route = pl.kernel(
    _route_body,
    out_type=[
        jax.ShapeDtypeStruct((2 * T,), jnp.float32),
        jax.ShapeDtypeStruct((2 * T,), jnp.int32),
    ],
    mesh=plsc.VectorSubcoreMesh(
        core_axis_name="c", subcore_axis_name="s",
        num_cores=NC, num_subcores=NS),
    scratch_types=[
        pltpu.VMEM((TPW * NUM_EXPERTS,), jnp.float32),
        pltpu.VMEM((2 * TPW,), jnp.float32),
        pltpu.VMEM((2 * TPW,), jnp.int32),
    ],
    compiler_params=pltpu.CompilerParams(needs_layout_passes=False),
)


@jax.jit
def kernel(hidden_states, gate_w, gate_b):
    b, s, h = hidden_states.shape
    t = b * s
    x = hidden_states.reshape(t, h)
    wt = gate_w.T

    logits = pl.pallas_call(
        _logits_block,
        grid=(t // BT,),
        in_specs=[
            pl.BlockSpec((BT, h), lambda i: (i, 0)),
            pl.BlockSpec((h, NUM_EXPERTS), lambda i: (0, 0)),
            pl.BlockSpec((NUM_EXPERTS,), lambda i: (0,)),
        ],
        out_specs=pl.BlockSpec((BT, NUM_EXPERTS), lambda i: (i, 0)),
        out_shape=jax.ShapeDtypeStruct((t, NUM_EXPERTS), jnp.float32),
    )(x, wt, gate_b)

    wf, ef = _route(logits.reshape(t * NUM_EXPERTS))
    weights = wf.reshape(t, TOP_K)
    experts = ef.reshape(t, TOP_K)
    aux_loss = jnp.array(0.0, dtype=jnp.float32)
    return (weights, experts, logits, aux_loss)


# speedup vs baseline: 1.2339x; 1.0777x over previous
import functools

import jax
import jax.numpy as jnp
from jax import lax
from jax.experimental import pallas as pl
from jax.experimental.pallas import tpu as pltpu
from jax.experimental.pallas import tpu_sc as plsc

HIDDEN = 768
NUM_EXPERTS = 64
TOP_K = 2
BT = 4096

T = 32768
NC, NS, L = 2, 16, 16
NW = NC * NS
TPW = T // NW
GROUPS = TPW // L


def _logits_block(x_ref, wt_ref, b_ref, logits_ref):
    logits_ref[...] = jax.lax.dot_general(
        x_ref[...], wt_ref[...], (((1,), (0,)), ((), ())),
        preferred_element_type=jnp.float32,
    ) + b_ref[...][None, :]


def _route_body(logits_hbm, w_hbm, e_hbm, chunk_v, w_v, e_v):
    cid = lax.axis_index("c")
    sid = lax.axis_index("s")
    wid = sid * NC + cid
    base = wid * TPW
    pltpu.sync_copy(
        logits_hbm.at[pl.ds(base * NUM_EXPERTS, TPW * NUM_EXPERTS)], chunk_v)

    lane = lax.iota(jnp.int32, L)

    def group(g, _):
        row = g * L + lane
        flat = row * NUM_EXPERTS
        neg = jnp.full((L,), -jnp.inf, jnp.float32)
        big = jnp.full((L,), NUM_EXPERTS, jnp.int32)

        m1, m2, a1, a2 = neg, neg, big, big
        for e in range(NUM_EXPERTS):
            ev = lane + e
            ev = jnp.where(ev >= NUM_EXPERTS, ev - NUM_EXPERTS, ev)
            v = plsc.load_gather(chunk_v, [flat + ev])
            b1 = (v > m1) | ((v == m1) & (ev < a1))
            b2 = (v > m2) | ((v == m2) & (ev < a2))
            a2 = jnp.where(b1, a1, jnp.where(b2, ev, a2))
            m2 = jnp.where(b1, m1, jnp.where(b2, v, m2))
            a1 = jnp.where(b1, ev, a1)
            m1 = jnp.where(b1, v, m1)
        w1 = 1.0 / (1.0 + jnp.exp(m2 - m1))
        w2 = 1.0 - w1
        pos = row * 2
        plsc.store_scatter(w_v, [pos], w1)
        plsc.store_scatter(w_v, [pos + 1], w2)
        plsc.store_scatter(e_v, [pos], a1)
        plsc.store_scatter(e_v, [pos + 1], a2)
        return 0

    lax.fori_loop(0, GROUPS, group, 0)
    pltpu.sync_copy(w_v, w_hbm.at[pl.ds(base * 2, 2 * TPW)])
    pltpu.sync_copy(e_v, e_hbm.at[pl.ds(base * 2, 2 * TPW)])


_route = pl.kernel(
    _route_body,
    out_type=[
        jax.ShapeDtypeStruct((2 * T,), jnp.float32),
        jax.ShapeDtypeStruct((2 * T,), jnp.int32),
    ],
    mesh=plsc.VectorSubcoreMesh(
        core_axis_name="c", subcore_axis_name="s",
        num_cores=NC, num_subcores=NS),
    scratch_types=[
        pltpu.VMEM((TPW * NUM_EXPERTS,), jnp.float32),
        pltpu.VMEM((2 * TPW,), jnp.float32),
        pltpu.VMEM((2 * TPW,), jnp.int32),
    ],
    compiler_params=pltpu.CompilerParams(needs_layout_passes=False),
)


@jax.jit
def kernel(hidden_states, gate_w, gate_b):
    b, s, h = hidden_states.shape
    t = b * s
    x = hidden_states.reshape(t, h)
    wt = gate_w.T

    logits = pl.pallas_call(
        _logits_block,
        grid=(t // BT,),
        in_specs=[
            pl.BlockSpec((BT, h), lambda i: (i, 0)),
            pl.BlockSpec((h, NUM_EXPERTS), lambda i: (0, 0)),
            pl.BlockSpec((NUM_EXPERTS,), lambda i: (0,)),
        ],
        out_specs=pl.BlockSpec((BT, NUM_EXPERTS), lambda i: (i, 0)),
        out_shape=jax.ShapeDtypeStruct((t, NUM_EXPERTS), jnp.float32),
    )(x, wt, gate_b)

    wf, ef = _route(logits.reshape(t * NUM_EXPERTS))
    weights = wf.reshape(t, TOP_K)
    experts = ef.reshape(t, TOP_K)
    aux_loss = jnp.array(0.0, dtype=jnp.float32)
    return (weights, experts, logits, aux_loss)


# speedup vs baseline: 1.4115x; 1.1439x over previous
import functools

import jax
import jax.numpy as jnp
from jax import lax
from jax.experimental import pallas as pl
from jax.experimental.pallas import tpu as pltpu
from jax.experimental.pallas import tpu_sc as plsc

HIDDEN = 768
NUM_EXPERTS = 64
TOP_K = 2
BT = 4096

T = 32768
NC, NS, L = 2, 16, 16
NW = NC * NS
TPW = T // NW
GROUPS = TPW // L


def _logits_block(x_ref, wt_ref, b_ref, logits_ref, logits_t_ref):
    logits = jax.lax.dot_general(
        x_ref[...], wt_ref[...], (((1,), (0,)), ((), ())),
        preferred_element_type=jnp.float32,
    ) + b_ref[...][None, :]
    logits_ref[...] = logits
    logits_t_ref[...] = logits.T


def _route_body(logits_t_hbm, w_hbm, e_hbm, chunk_v, w_v, e_v):
    cid = lax.axis_index("c")
    sid = lax.axis_index("s")
    wid = sid * NC + cid
    base = wid * TPW
    pltpu.sync_copy(logits_t_hbm.at[:, pl.ds(base, TPW)], chunk_v)

    lane = lax.iota(jnp.int32, L)

    def group(g, _):
        row = g * L + lane
        neg = jnp.full((L,), -jnp.inf, jnp.float32)
        big = jnp.full((L,), NUM_EXPERTS, jnp.int32)

        m1, m2, a1, a2 = neg, neg, big, big
        for e in range(NUM_EXPERTS):
            ev = jnp.full((L,), e, jnp.int32)
            v = chunk_v[e, pl.ds(g * L, L)]
            gt1 = v > m1
            gt2 = v > m2
            a2 = jnp.where(gt1, a1, jnp.where(gt2, ev, a2))
            m2 = jnp.where(gt1, m1, jnp.where(gt2, v, m2))
            a1 = jnp.where(gt1, ev, a1)
            m1 = jnp.where(gt1, v, m1)
        w1 = 1.0 / (1.0 + jnp.exp(m2 - m1))
        w2 = 1.0 - w1
        pos = row * 2
        plsc.store_scatter(w_v, [pos], w1)
        plsc.store_scatter(w_v, [pos + 1], w2)
        plsc.store_scatter(e_v, [pos], a1)
        plsc.store_scatter(e_v, [pos + 1], a2)
        return 0

    lax.fori_loop(0, GROUPS, group, 0)
    pltpu.sync_copy(w_v, w_hbm.at[pl.ds(base * 2, 2 * TPW)])
    pltpu.sync_copy(e_v, e_hbm.at[pl.ds(base * 2, 2 * TPW)])


_route = pl.kernel(
    _route_body,
    out_type=[
        jax.ShapeDtypeStruct((2 * T,), jnp.float32),
        jax.ShapeDtypeStruct((2 * T,), jnp.int32),
    ],
    mesh=plsc.VectorSubcoreMesh(
        core_axis_name="c", subcore_axis_name="s",
        num_cores=NC, num_subcores=NS),
    scratch_types=[
        pltpu.VMEM((NUM_EXPERTS, TPW), jnp.float32),
        pltpu.VMEM((2 * TPW,), jnp.float32),
        pltpu.VMEM((2 * TPW,), jnp.int32),
    ],
    compiler_params=pltpu.CompilerParams(needs_layout_passes=False),
)


@jax.jit
def kernel(hidden_states, gate_w, gate_b):
    b, s, h = hidden_states.shape
    t = b * s
    x = hidden_states.reshape(t, h)
    wt = gate_w.T

    logits, logits_t = pl.pallas_call(
        _logits_block,
        grid=(t // BT,),
        in_specs=[
            pl.BlockSpec((BT, h), lambda i: (i, 0)),
            pl.BlockSpec((h, NUM_EXPERTS), lambda i: (0, 0)),
            pl.BlockSpec((NUM_EXPERTS,), lambda i: (0,)),
        ],
        out_specs=[
            pl.BlockSpec((BT, NUM_EXPERTS), lambda i: (i, 0)),
            pl.BlockSpec((NUM_EXPERTS, BT), lambda i: (0, i)),
        ],
        out_shape=[
            jax.ShapeDtypeStruct((t, NUM_EXPERTS), jnp.float32),
            jax.ShapeDtypeStruct((NUM_EXPERTS, t), jnp.float32),
        ],
    )(x, wt, gate_b)

    wf, ef = _route(logits_t)
    weights = wf.reshape(t, TOP_K)
    experts = ef.reshape(t, TOP_K)
    aux_loss = jnp.array(0.0, dtype=jnp.float32)
    return (weights, experts, logits, aux_loss)


# speedup vs baseline: 1.4282x; 1.0118x over previous
import functools

import jax
import jax.numpy as jnp
from jax import lax
from jax.experimental import pallas as pl
from jax.experimental.pallas import tpu as pltpu
from jax.experimental.pallas import tpu_sc as plsc

HIDDEN = 768
NUM_EXPERTS = 64
TOP_K = 2
BT = 4096

T = 32768
NC, NS, L = 2, 16, 16
NW = NC * NS
TPW = T // NW
GROUPS = TPW // L


def _logits_block(x_ref, wt_ref, b_ref, logits_ref, logits_t_ref):
    logits = jax.lax.dot_general(
        x_ref[...], wt_ref[...], (((1,), (0,)), ((), ())),
        preferred_element_type=jnp.float32,
    ) + b_ref[...][None, :]
    logits_ref[...] = logits
    logits_t_ref[...] = logits.T


def _route_body(logits_t_hbm, w_hbm, e_hbm, chunk_v, w_v, e_v):
    cid = lax.axis_index("c")
    sid = lax.axis_index("s")
    wid = sid * NC + cid
    base = wid * TPW
    pltpu.sync_copy(logits_t_hbm.at[:, pl.ds(base, TPW)], chunk_v)

    lane = lax.iota(jnp.int32, L)
    GI = 4

    def group(g0, _):
        neg = jnp.full((L,), -jnp.inf, jnp.float32)
        big = jnp.full((L,), NUM_EXPERTS, jnp.int32)

        st = [[neg, neg, big, big] for _ in range(GI)]
        for e in range(NUM_EXPERTS):
            ev = jnp.full((L,), e, jnp.int32)
            for j in range(GI):
                m1, m2, a1, a2 = st[j]
                v = chunk_v[e, pl.ds((g0 + j) * L, L)]
                gt1 = v > m1
                gt2 = v > m2
                a2 = jnp.where(gt1, a1, jnp.where(gt2, ev, a2))
                m2 = jnp.where(gt1, m1, jnp.where(gt2, v, m2))
                a1 = jnp.where(gt1, ev, a1)
                m1 = jnp.where(gt1, v, m1)
                st[j] = [m1, m2, a1, a2]
        for j in range(GI):
            m1, m2, a1, a2 = st[j]
            w1 = 1.0 / (1.0 + jnp.exp(m2 - m1))
            w2 = 1.0 - w1
            pos = ((g0 + j) * L + lane) * 2
            plsc.store_scatter(w_v, [pos], w1)
            plsc.store_scatter(w_v, [pos + 1], w2)
            plsc.store_scatter(e_v, [pos], a1)
            plsc.store_scatter(e_v, [pos + 1], a2)
        return 0

    lax.fori_loop(0, GROUPS // GI, lambda i, c: group(i * GI, c), 0)
    pltpu.sync_copy(w_v, w_hbm.at[pl.ds(base * 2, 2 * TPW)])
    pltpu.sync_copy(e_v, e_hbm.at[pl.ds(base * 2, 2 * TPW)])


_route = pl.kernel(
    _route_body,
    out_type=[
        jax.ShapeDtypeStruct((2 * T,), jnp.float32),
        jax.ShapeDtypeStruct((2 * T,), jnp.int32),
    ],
    mesh=plsc.VectorSubcoreMesh(
        core_axis_name="c", subcore_axis_name="s",
        num_cores=NC, num_subcores=NS),
    scratch_types=[
        pltpu.VMEM((NUM_EXPERTS, TPW), jnp.float32),
        pltpu.VMEM((2 * TPW,), jnp.float32),
        pltpu.VMEM((2 * TPW,), jnp.int32),
    ],
    compiler_params=pltpu.CompilerParams(needs_layout_passes=False),
)


@jax.jit
def kernel(hidden_states, gate_w, gate_b):
    b, s, h = hidden_states.shape
    t = b * s
    x = hidden_states.reshape(t, h)
    wt = gate_w.T

    logits, logits_t = pl.pallas_call(
        _logits_block,
        grid=(t // BT,),
        in_specs=[
            pl.BlockSpec((BT, h), lambda i: (i, 0)),
            pl.BlockSpec((h, NUM_EXPERTS), lambda i: (0, 0)),
            pl.BlockSpec((NUM_EXPERTS,), lambda i: (0,)),
        ],
        out_specs=[
            pl.BlockSpec((BT, NUM_EXPERTS), lambda i: (i, 0)),
            pl.BlockSpec((NUM_EXPERTS, BT), lambda i: (0, i)),
        ],
        out_shape=[
            jax.ShapeDtypeStruct((t, NUM_EXPERTS), jnp.float32),
            jax.ShapeDtypeStruct((NUM_EXPERTS, t), jnp.float32),
        ],
    )(x, wt, gate_b)

    wf, ef = _route(logits_t)
    weights = wf.reshape(t, TOP_K)
    experts = ef.reshape(t, TOP_K)
    aux_loss = jnp.array(0.0, dtype=jnp.float32)
    return (weights, experts, logits, aux_loss)
